# Initial kernel scaffold; baseline (speedup 1.0000x reference)
import jax
import jax.numpy as jnp
from jax.experimental import pallas as pl


def kernel(x, edge_index, edge_weight, W1, b1, W2, b2, Wd1, bd1, Wd2, bd2):
    raise NotImplementedError("write your pallas kernel here")


# speedup vs baseline: 5.2478x; 5.2478x over previous
import functools

import jax
import jax.numpy as jnp
from jax import lax
from jax.experimental import pallas as pl
from jax.experimental.pallas import tpu as pltpu
from jax.experimental.pallas import tpu_sc as plsc

N = 10000
E = 160000
IN_CH = 256
HID_CH = 512
LAT_CH = 256

NCORE = 2
NSUB = 16
LANES = 16
EB = 128

E_TILE_AGG = 10240
NB_AGG = E_TILE_AGG // EB
E_PAD = NSUB * E_TILE_AGG
E_TILE_DEG = E_PAD // (NCORE * NSUB)
NB_DEG = E_TILE_DEG // EB
DEG_SLICE = 640
DEG_PAD = NSUB * DEG_SLICE
N_ACC = DEG_PAD
NROW = N_ACC // NSUB

_MESH = dict(core_axis_name="c", subcore_axis_name="s", num_cores=NCORE,
             num_subcores=NSUB)


@functools.partial(
    pl.kernel,
    out_type=jax.ShapeDtypeStruct((NCORE, DEG_PAD), jnp.float32),
    mesh=plsc.VectorSubcoreMesh(**_MESH),
    scratch_types=[
        pltpu.VMEM((NB_DEG, EB), jnp.int32),
        pltpu.VMEM((NB_DEG, EB), jnp.float32),
        pltpu.VMEM((DEG_SLICE,), jnp.float32),
        pltpu.VMEM_SHARED((DEG_PAD,), jnp.float32),
    ],
)
def _deg_kernel(dst_hbm, ew_hbm, out_hbm, dstv, ewv, zv, deg_sh):
    cid = lax.axis_index("c")
    sid = lax.axis_index("s")
    w = cid * NSUB + sid

    def z1(i, _):
        zv[pl.ds(i * LANES, LANES)] = jnp.zeros((LANES,), jnp.float32)
        return 0
    lax.fori_loop(0, DEG_SLICE // LANES, z1, 0)
    pltpu.sync_copy(zv, deg_sh.at[pl.ds(sid * DEG_SLICE, DEG_SLICE)])
    pltpu.sync_copy(dst_hbm.at[w], dstv)
    pltpu.sync_copy(ew_hbm.at[w], ewv)
    plsc.subcore_barrier()

    def body(b, _):
        pltpu.sync_copy(ewv.at[b], deg_sh.at[dstv.at[b]], add=True)
        return 0
    lax.fori_loop(0, NB_DEG, body, 0)
    plsc.subcore_barrier()
    pltpu.sync_copy(deg_sh.at[pl.ds(sid * DEG_SLICE, DEG_SLICE)],
                    out_hbm.at[cid, pl.ds(sid * DEG_SLICE, DEG_SLICE)])


def _make_agg(n_chunks):
    cpc = n_chunks // NCORE

    @functools.partial(
        pl.kernel,
        out_type=jax.ShapeDtypeStruct((n_chunks, N_ACC, 128), jnp.float32),
        mesh=plsc.VectorSubcoreMesh(**_MESH),
        scratch_types=[
            pltpu.VMEM((NB_AGG, EB), jnp.int32),
            pltpu.VMEM((NB_AGG, EB), jnp.int32),
            pltpu.VMEM((NB_AGG, EB), jnp.float32),
            pltpu.VMEM((EB, 128), jnp.float32),
            pltpu.VMEM_SHARED((N_ACC, 128), jnp.float32),
            pltpu.SemaphoreType.DMA,
        ],
    )
    def agg(hp_hbm, dst_hbm, src_hbm, ew_hbm, out_hbm,
            dstv, offv, ewv, rows, acc_sh, sem):
        cid = lax.axis_index("c")
        sid = lax.axis_index("s")
        pltpu.sync_copy(dst_hbm.at[sid], dstv)
        pltpu.sync_copy(src_hbm.at[sid], offv)
        pltpu.sync_copy(ew_hbm.at[sid], ewv)

        for cl in range(cpc):
            chunk = cid * cpc + cl
            delta = cid * cpc * N if cl == 0 else N

            def o1(i, _):
                for j in range(EB // LANES):
                    offv[i, pl.ds(j * LANES, LANES)] = (
                        offv[i, pl.ds(j * LANES, LANES)] + delta)
                return 0
            lax.fori_loop(0, NB_AGG, o1, 0)

            def zf(i, _):
                for j in range(8):
                    rows[i, pl.ds(j * LANES, LANES)] = jnp.zeros(
                        (LANES,), jnp.float32)
                return 0
            lax.fori_loop(0, EB, zf, 0)

            def zc(k, _):
                pltpu.sync_copy(rows, acc_sh.at[pl.ds(sid * NROW + k * 128, 128)])
                return 0
            lax.fori_loop(0, 5, zc, 0)
            plsc.subcore_barrier()

            def eb(b, _):
                pltpu.async_copy(hp_hbm.at[offv.at[b]], rows, sem).wait()

                def sc1(g, _):
                    ew16 = ewv[b, pl.ds(g * LANES, LANES)]
                    for t in range(LANES):
                        s = ew16[t]
                        e = g * LANES + t
                        for j in range(8):
                            rows[e, pl.ds(j * LANES, LANES)] = (
                                rows[e, pl.ds(j * LANES, LANES)] * s)
                    return 0
                lax.fori_loop(0, EB // LANES, sc1, 0)
                pltpu.sync_copy(rows, acc_sh.at[dstv.at[b]], add=True)
                return 0
            lax.fori_loop(0, NB_AGG, eb, 0)
            plsc.subcore_barrier()
            pltpu.sync_copy(acc_sh.at[pl.ds(sid * NROW, NROW)],
                            out_hbm.at[chunk, pl.ds(sid * NROW, NROW)])
    return agg


_agg4 = _make_agg(4)
_agg2 = _make_agg(2)


_RB = 1000
_GRID = N // _RB


def _tc1_body(x_ref, deg_ref, w1_ref, hp_ref, dinv_ref):
    deg = deg_ref[...]
    dinv = jnp.where(deg > 0, lax.rsqrt(deg), 0.0)
    p = jnp.dot(x_ref[...], w1_ref[...], preferred_element_type=jnp.float32)
    hp_ref[...] = p * dinv
    dinv_ref[...] = dinv


def _tc2_body(agg_ref, hp_ref, dinv_ref, b1_ref, w2_ref, out_ref):
    dinv = dinv_ref[...]
    h = jnp.maximum(dinv * (agg_ref[...] + hp_ref[...]) + b1_ref[...], 0.0)
    out_ref[...] = dinv * jnp.dot(h, w2_ref[...],
                                  preferred_element_type=jnp.float32)


def _tc3_body(agg_ref, hp_ref, dinv_ref, b2_ref, wd1_ref, bd1_ref, wd2_ref,
              bd2_ref, out_ref):
    dinv = dinv_ref[...]
    z = jnp.maximum(dinv * (agg_ref[...] + hp_ref[...]) + b2_ref[...], 0.0)
    d = jnp.maximum(jnp.dot(z, wd1_ref[...],
                            preferred_element_type=jnp.float32) + bd1_ref[...],
                    0.0)
    out_ref[...] = jax.nn.sigmoid(
        jnp.dot(d, wd2_ref[...], preferred_element_type=jnp.float32)
        + bd2_ref[...])


def _row_spec(ch):
    return pl.BlockSpec((_RB, ch), lambda i: (i, 0))


def _full_spec(r, c):
    return pl.BlockSpec((r, c), lambda i: (0, 0))


_tc1 = pl.pallas_call(
    _tc1_body,
    grid=(_GRID,),
    in_specs=[_row_spec(IN_CH), _row_spec(1), _full_spec(IN_CH, HID_CH)],
    out_specs=[_row_spec(HID_CH), _row_spec(1)],
    out_shape=[jax.ShapeDtypeStruct((N, HID_CH), jnp.float32),
               jax.ShapeDtypeStruct((N, 1), jnp.float32)],
)

_tc2 = pl.pallas_call(
    _tc2_body,
    grid=(_GRID,),
    in_specs=[_row_spec(HID_CH), _row_spec(HID_CH), _row_spec(1),
              _full_spec(1, HID_CH), _full_spec(HID_CH, LAT_CH)],
    out_specs=_row_spec(LAT_CH),
    out_shape=jax.ShapeDtypeStruct((N, LAT_CH), jnp.float32),
)

_tc3 = pl.pallas_call(
    _tc3_body,
    grid=(_GRID,),
    in_specs=[_row_spec(LAT_CH), _row_spec(LAT_CH), _row_spec(1),
              _full_spec(1, LAT_CH), _full_spec(LAT_CH, HID_CH),
              _full_spec(1, HID_CH), _full_spec(HID_CH, IN_CH),
              _full_spec(1, IN_CH)],
    out_specs=_row_spec(IN_CH),
    out_shape=jax.ShapeDtypeStruct((N, IN_CH), jnp.float32),
)


def _to_chunks(a, n_chunks):
    return a.reshape(N, n_chunks, 128).transpose(1, 0, 2).reshape(
        n_chunks * N, 128)


def _from_chunks(a, n_chunks):
    return a[:, :N, :].transpose(1, 0, 2).reshape(N, n_chunks * 128)


def kernel(x, edge_index, edge_weight, W1, b1, W2, b2, Wd1, bd1, Wd2, bd2):
    src = edge_index[0].astype(jnp.int32)
    dst = edge_index[1].astype(jnp.int32)
    ew = edge_weight.astype(jnp.float32)
    pad = E_PAD - E
    src_p = jnp.concatenate([src, jnp.zeros((pad,), jnp.int32)])
    dst_p = jnp.concatenate([dst, jnp.zeros((pad,), jnp.int32)])
    ew_p = jnp.concatenate([ew, jnp.zeros((pad,), jnp.float32)])

    deg_part = _deg_kernel(dst_p.reshape(NCORE * NSUB, NB_DEG, EB),
                           ew_p.reshape(NCORE * NSUB, NB_DEG, EB))
    deg = (deg_part[0, :N] + deg_part[1, :N] + 1.0).reshape(N, 1)

    src_a = src_p.reshape(NSUB, NB_AGG, EB)
    dst_a = dst_p.reshape(NSUB, NB_AGG, EB)
    ew_a = ew_p.reshape(NSUB, NB_AGG, EB)

    h1p, dinv = _tc1(x, deg, W1)
    agg1 = _from_chunks(_agg4(_to_chunks(h1p, 4), dst_a, src_a, ew_a), 4)
    h2p = _tc2(agg1, h1p, dinv, b1.reshape(1, HID_CH), W2)
    agg2 = _from_chunks(_agg2(_to_chunks(h2p, 2), dst_a, src_a, ew_a), 2)
    x_hat = _tc3(agg2, h2p, dinv, b2.reshape(1, LAT_CH), Wd1,
                 bd1.reshape(1, HID_CH), Wd2, bd2.reshape(1, IN_CH))
    return x_hat


# speedup vs baseline: 7.1358x; 1.3598x over previous
import functools

import jax
import jax.numpy as jnp
from jax import lax
from jax.experimental import pallas as pl
from jax.experimental.pallas import tpu as pltpu
from jax.experimental.pallas import tpu_sc as plsc

N = 10000
E = 160000
IN_CH = 256
HID_CH = 512
LAT_CH = 256

NCORE = 2
NSUB = 16
LANES = 16
EB = 128

AB = 64
E_TILE_AGG = 10240
NB_AGG = E_TILE_AGG // AB
PACK_SHIFT = 14
E_PAD = NSUB * E_TILE_AGG
E_TILE_DEG = E_PAD // (NCORE * NSUB)
NB_DEG = E_TILE_DEG // EB
DEG_SLICE = 640
DEG_PAD = NSUB * DEG_SLICE
N_ACC = DEG_PAD
NROW = N_ACC // NSUB

_MESH = dict(core_axis_name="c", subcore_axis_name="s", num_cores=NCORE,
             num_subcores=NSUB)


@functools.partial(
    pl.kernel,
    out_type=jax.ShapeDtypeStruct((NCORE, DEG_PAD), jnp.float32),
    mesh=plsc.VectorSubcoreMesh(**_MESH),
    scratch_types=[
        pltpu.VMEM((NB_DEG, EB), jnp.int32),
        pltpu.VMEM((NB_DEG, EB), jnp.float32),
        pltpu.VMEM((DEG_SLICE,), jnp.float32),
        pltpu.VMEM_SHARED((DEG_PAD,), jnp.float32),
    ],
)
def _deg_kernel(dst_hbm, ew_hbm, out_hbm, dstv, ewv, zv, deg_sh):
    cid = lax.axis_index("c")
    sid = lax.axis_index("s")
    w = cid * NSUB + sid

    def z1(i, _):
        zv[pl.ds(i * LANES, LANES)] = jnp.zeros((LANES,), jnp.float32)
        return 0
    lax.fori_loop(0, DEG_SLICE // LANES, z1, 0)
    pltpu.sync_copy(zv, deg_sh.at[pl.ds(sid * DEG_SLICE, DEG_SLICE)])
    pltpu.sync_copy(dst_hbm.at[w], dstv)
    pltpu.sync_copy(ew_hbm.at[w], ewv)
    plsc.subcore_barrier()

    def body(b, _):
        pltpu.sync_copy(ewv.at[b], deg_sh.at[dstv.at[b]], add=True)
        return 0
    lax.fori_loop(0, NB_DEG, body, 0)
    plsc.subcore_barrier()
    pltpu.sync_copy(deg_sh.at[pl.ds(sid * DEG_SLICE, DEG_SLICE)],
                    out_hbm.at[cid, pl.ds(sid * DEG_SLICE, DEG_SLICE)])


def _make_agg(n_chunks):
    cpc = n_chunks // NCORE
    assert NB_AGG % 3 == 1 and NB_AGG >= 7

    @functools.partial(
        pl.kernel,
        out_type=jax.ShapeDtypeStruct((n_chunks, N_ACC, 128), jnp.float32),
        mesh=plsc.VectorSubcoreMesh(**_MESH),
        scratch_types=[
            pltpu.VMEM((NB_AGG // 2, 2 * AB), jnp.int32),
            pltpu.VMEM((NB_AGG // 2, 2 * AB), jnp.float32),
            pltpu.VMEM((AB, 128), jnp.float32),
            pltpu.VMEM((AB, 128), jnp.float32),
            pltpu.VMEM((AB, 128), jnp.float32),
            pltpu.VMEM((1, AB), jnp.int32),
            pltpu.VMEM((1, AB), jnp.int32),
            pltpu.VMEM((1, AB), jnp.int32),
            pltpu.VMEM((1, AB), jnp.int32),
            pltpu.VMEM((1, AB), jnp.int32),
            pltpu.VMEM((1, AB), jnp.int32),
            pltpu.VMEM_SHARED((N_ACC, 128), jnp.float32),
            pltpu.SemaphoreType.DMA,
            pltpu.SemaphoreType.DMA,
            pltpu.SemaphoreType.DMA,
            pltpu.SemaphoreType.DMA,
            pltpu.SemaphoreType.DMA,
            pltpu.SemaphoreType.DMA,
            pltpu.SemaphoreType.DMA,
        ],
    )
    def agg(hp_hbm, pe_hbm, ew_hbm, out_hbm,
            pev, ewv, r0, r1, r2, g0, g1, g2, s0, s1, s2, acc_sh,
            gA, gB, gC, sA, sB, sC, zsem):
        rows = [r0, r1, r2]
        gidx = [g0, g1, g2]
        sidx = [s0, s1, s2]
        gsem = [gA, gB, gC]
        ssem = [sA, sB, sC]
        cid = lax.axis_index("c")
        sid = lax.axis_index("s")
        pltpu.sync_copy(pe_hbm.at[sid], pev)
        pltpu.sync_copy(ew_hbm.at[sid], ewv)
        mask = (1 << PACK_SHIFT) - 1

        def fire_gather(b, k, base):
            b2, bc = b // 2, (b % 2) * AB

            def u(g, _):
                v = pev[b2, pl.ds(bc + g * LANES, LANES)]
                gidx[k][0, pl.ds(g * LANES, LANES)] = (v & mask) + base
                return 0
            lax.fori_loop(0, AB // LANES, u, 0)
            pltpu.async_copy(hp_hbm.at[gidx[k].at[0]], rows[k], gsem[k])

        def wait_gather(k):
            pltpu.make_async_copy(hp_hbm.at[gidx[k].at[0]], rows[k],
                                  gsem[k]).wait()

        def scale(b, k):
            b2, bc = b // 2, (b % 2) * AB

            def sc1(g, _):
                ew16 = ewv[b2, pl.ds(bc + g * LANES, LANES)]
                for t in range(LANES):
                    s = ew16[t]
                    e = g * LANES + t
                    for j in range(8):
                        rows[k][e, pl.ds(j * LANES, LANES)] = (
                            rows[k][e, pl.ds(j * LANES, LANES)] * s)
                return 0
            lax.fori_loop(0, AB // LANES, sc1, 0)

        def fire_scatter(b, k):
            b2, bc = b // 2, (b % 2) * AB

            def u(g, _):
                v = pev[b2, pl.ds(bc + g * LANES, LANES)]
                sidx[k][0, pl.ds(g * LANES, LANES)] = lax.shift_right_logical(
                    v, PACK_SHIFT)
                return 0
            lax.fori_loop(0, AB // LANES, u, 0)
            pltpu.async_copy(rows[k], acc_sh.at[sidx[k].at[0]], ssem[k],
                             add=True)

        def wait_scatter(k):
            pltpu.make_async_copy(rows[k], acc_sh.at[sidx[k].at[0]],
                                  ssem[k]).wait()

        for cl in range(cpc):
            chunk = cid * cpc + cl
            base = chunk * N

            def zf(i, _):
                for j in range(8):
                    r0[i, pl.ds(j * LANES, LANES)] = jnp.zeros(
                        (LANES,), jnp.float32)
                return 0
            lax.fori_loop(0, AB, zf, 0)

            def zc(q, _):
                pltpu.async_copy(r0, acc_sh.at[pl.ds(sid * NROW + q * AB, AB)],
                                 zsem)
                return 0
            lax.fori_loop(0, NROW // AB, zc, 0)

            def zw(q, _):
                pltpu.make_async_copy(r0, acc_sh.at[pl.ds(sid * NROW, AB)],
                                      zsem).wait()
                return 0
            lax.fori_loop(0, NROW // AB, zw, 0)
            plsc.subcore_barrier()

            fire_gather(0, 0, base)
            fire_gather(1, 1, base)
            wait_gather(0)
            scale(0, 0)
            fire_scatter(0, 0)
            fire_gather(2, 2, base)
            wait_gather(1)
            scale(1, 1)
            fire_scatter(1, 1)

            def main(i, _):
                for j in range(3):
                    b = 2 + i * 3 + j
                    k = (2 + j) % 3
                    kp = j
                    wait_scatter(kp)
                    fire_gather(b + 1, kp, base)
                    wait_gather(k)
                    scale(b, k)
                    fire_scatter(b, k)
                return 0
            lax.fori_loop(0, (NB_AGG - 4) // 3, main, 0)

            b = NB_AGG - 2
            wait_scatter(0)
            fire_gather(b + 1, 0, base)
            wait_gather(2)
            scale(b, 2)
            fire_scatter(b, 2)
            b = NB_AGG - 1
            wait_scatter(1)
            wait_gather(0)
            scale(b, 0)
            fire_scatter(b, 0)
            wait_scatter(2)
            wait_scatter(0)
            plsc.subcore_barrier()
            pltpu.sync_copy(acc_sh.at[pl.ds(sid * NROW, NROW)],
                            out_hbm.at[chunk, pl.ds(sid * NROW, NROW)])
    return agg


_agg4 = _make_agg(4)
_agg2 = _make_agg(2)


_RB = 1000
_GRID = N // _RB


def _tc1_body(x_ref, deg_ref, w1_ref, hp_ref, dinv_ref):
    deg = deg_ref[...]
    dinv = jnp.where(deg > 0, lax.rsqrt(deg), 0.0)
    p = jnp.dot(x_ref[...], w1_ref[...], preferred_element_type=jnp.float32)
    hp_ref[...] = p * dinv
    dinv_ref[...] = dinv


def _tc2_body(agg_ref, hp_ref, dinv_ref, b1_ref, w2_ref, out_ref):
    dinv = dinv_ref[...]
    h = jnp.maximum(dinv * (agg_ref[...] + hp_ref[...]) + b1_ref[...], 0.0)
    out_ref[...] = dinv * jnp.dot(h, w2_ref[...],
                                  preferred_element_type=jnp.float32)


def _tc3_body(agg_ref, hp_ref, dinv_ref, b2_ref, wd1_ref, bd1_ref, wd2_ref,
              bd2_ref, out_ref):
    dinv = dinv_ref[...]
    z = jnp.maximum(dinv * (agg_ref[...] + hp_ref[...]) + b2_ref[...], 0.0)
    d = jnp.maximum(jnp.dot(z, wd1_ref[...],
                            preferred_element_type=jnp.float32) + bd1_ref[...],
                    0.0)
    out_ref[...] = jax.nn.sigmoid(
        jnp.dot(d, wd2_ref[...], preferred_element_type=jnp.float32)
        + bd2_ref[...])


def _row_spec(ch):
    return pl.BlockSpec((_RB, ch), lambda i: (i, 0))


def _full_spec(r, c):
    return pl.BlockSpec((r, c), lambda i: (0, 0))


_tc1 = pl.pallas_call(
    _tc1_body,
    grid=(_GRID,),
    in_specs=[_row_spec(IN_CH), _row_spec(1), _full_spec(IN_CH, HID_CH)],
    out_specs=[_row_spec(HID_CH), _row_spec(1)],
    out_shape=[jax.ShapeDtypeStruct((N, HID_CH), jnp.float32),
               jax.ShapeDtypeStruct((N, 1), jnp.float32)],
)

_tc2 = pl.pallas_call(
    _tc2_body,
    grid=(_GRID,),
    in_specs=[_row_spec(HID_CH), _row_spec(HID_CH), _row_spec(1),
              _full_spec(1, HID_CH), _full_spec(HID_CH, LAT_CH)],
    out_specs=_row_spec(LAT_CH),
    out_shape=jax.ShapeDtypeStruct((N, LAT_CH), jnp.float32),
)

_tc3 = pl.pallas_call(
    _tc3_body,
    grid=(_GRID,),
    in_specs=[_row_spec(LAT_CH), _row_spec(LAT_CH), _row_spec(1),
              _full_spec(1, LAT_CH), _full_spec(LAT_CH, HID_CH),
              _full_spec(1, HID_CH), _full_spec(HID_CH, IN_CH),
              _full_spec(1, IN_CH)],
    out_specs=_row_spec(IN_CH),
    out_shape=jax.ShapeDtypeStruct((N, IN_CH), jnp.float32),
)


def _to_chunks(a, n_chunks):
    return a.reshape(N, n_chunks, 128).transpose(1, 0, 2).reshape(
        n_chunks * N, 128)


def _from_chunks(a, n_chunks):
    return a[:, :N, :].transpose(1, 0, 2).reshape(N, n_chunks * 128)


def kernel(x, edge_index, edge_weight, W1, b1, W2, b2, Wd1, bd1, Wd2, bd2):
    src = edge_index[0].astype(jnp.int32)
    dst = edge_index[1].astype(jnp.int32)
    ew = edge_weight.astype(jnp.float32)
    pad = E_PAD - E
    src_p = jnp.concatenate([src, jnp.zeros((pad,), jnp.int32)])
    dst_p = jnp.concatenate([dst, jnp.zeros((pad,), jnp.int32)])
    ew_p = jnp.concatenate([ew, jnp.zeros((pad,), jnp.float32)])

    deg_part = _deg_kernel(dst_p.reshape(NCORE * NSUB, NB_DEG, EB),
                           ew_p.reshape(NCORE * NSUB, NB_DEG, EB))
    deg = (deg_part[0, :N] + deg_part[1, :N] + 1.0).reshape(N, 1)

    pe_a = (src_p + dst_p * (1 << PACK_SHIFT)).reshape(NSUB, NB_AGG // 2,
                                                       2 * AB)
    ew_a = ew_p.reshape(NSUB, NB_AGG // 2, 2 * AB)

    h1p, dinv = _tc1(x, deg, W1)
    agg1 = _from_chunks(_agg4(_to_chunks(h1p, 4), pe_a, ew_a), 4)
    h2p = _tc2(agg1, h1p, dinv, b1.reshape(1, HID_CH), W2)
    agg2 = _from_chunks(_agg2(_to_chunks(h2p, 2), pe_a, ew_a), 2)
    x_hat = _tc3(agg2, h2p, dinv, b2.reshape(1, LAT_CH), Wd1,
                 bd1.reshape(1, HID_CH), Wd2, bd2.reshape(1, IN_CH))
    return x_hat


# speedup vs baseline: 8.1392x; 1.1406x over previous
import functools

import jax
import jax.numpy as jnp
from jax import lax
from jax.experimental import pallas as pl
from jax.experimental.pallas import tpu as pltpu
from jax.experimental.pallas import tpu_sc as plsc

N = 10000
E = 160000
IN_CH = 256
HID_CH = 512
LAT_CH = 256

NCORE = 2
NSUB = 16
LANES = 16
EB = 128

AB = 112
NB_AGG = 91
E_TILE_AGG = NB_AGG * AB
E_PAD_AGG = NSUB * E_TILE_AGG
PACK_SHIFT = 14
E_TILE_DEG = 5120
E_PAD_DEG = E_TILE_DEG * NCORE * NSUB
NB_DEG = E_TILE_DEG // EB
DEG_SLICE = 640
DEG_PAD = NSUB * DEG_SLICE
N_ACC = DEG_PAD
NROW = N_ACC // NSUB

_MESH = dict(core_axis_name="c", subcore_axis_name="s", num_cores=NCORE,
             num_subcores=NSUB)


@functools.partial(
    pl.kernel,
    out_type=jax.ShapeDtypeStruct((NCORE, DEG_PAD), jnp.float32),
    mesh=plsc.VectorSubcoreMesh(**_MESH),
    scratch_types=[
        pltpu.VMEM((NB_DEG, EB), jnp.int32),
        pltpu.VMEM((NB_DEG, EB), jnp.float32),
        pltpu.VMEM((DEG_SLICE,), jnp.float32),
        pltpu.VMEM_SHARED((DEG_PAD,), jnp.float32),
    ],
)
def _deg_kernel(dst_hbm, ew_hbm, out_hbm, dstv, ewv, zv, deg_sh):
    cid = lax.axis_index("c")
    sid = lax.axis_index("s")
    w = cid * NSUB + sid

    def z1(i, _):
        zv[pl.ds(i * LANES, LANES)] = jnp.zeros((LANES,), jnp.float32)
        return 0
    lax.fori_loop(0, DEG_SLICE // LANES, z1, 0)
    pltpu.sync_copy(zv, deg_sh.at[pl.ds(sid * DEG_SLICE, DEG_SLICE)])
    pltpu.sync_copy(dst_hbm.at[w], dstv)
    pltpu.sync_copy(ew_hbm.at[w], ewv)
    plsc.subcore_barrier()

    def body(b, _):
        pltpu.sync_copy(ewv.at[b], deg_sh.at[dstv.at[b]], add=True)
        return 0
    lax.fori_loop(0, NB_DEG, body, 0)
    plsc.subcore_barrier()
    pltpu.sync_copy(deg_sh.at[pl.ds(sid * DEG_SLICE, DEG_SLICE)],
                    out_hbm.at[cid, pl.ds(sid * DEG_SLICE, DEG_SLICE)])


def _make_agg(n_chunks):
    cpc = n_chunks // NCORE
    assert NB_AGG % 3 == 1 and NB_AGG >= 7

    @functools.partial(
        pl.kernel,
        out_type=jax.ShapeDtypeStruct((n_chunks, N_ACC, 128), jnp.float32),
        mesh=plsc.VectorSubcoreMesh(**_MESH),
        scratch_types=[
            pltpu.VMEM((AB, 128), jnp.float32),
            pltpu.VMEM((AB, 128), jnp.float32),
            pltpu.VMEM((AB, 128), jnp.float32),
            pltpu.VMEM((1, AB), jnp.int32),
            pltpu.VMEM((1, AB), jnp.int32),
            pltpu.VMEM((1, AB), jnp.int32),
            pltpu.VMEM((1, AB), jnp.float32),
            pltpu.VMEM((1, AB), jnp.float32),
            pltpu.VMEM((1, AB), jnp.float32),
            pltpu.VMEM((1, AB), jnp.int32),
            pltpu.VMEM((1, AB), jnp.int32),
            pltpu.VMEM((1, AB), jnp.int32),
            pltpu.VMEM_SHARED((N_ACC, 128), jnp.float32),
            pltpu.SemaphoreType.DMA,
            pltpu.SemaphoreType.DMA,
            pltpu.SemaphoreType.DMA,
            pltpu.SemaphoreType.DMA,
            pltpu.SemaphoreType.DMA,
            pltpu.SemaphoreType.DMA,
            pltpu.SemaphoreType.DMA,
            pltpu.SemaphoreType.DMA,
            pltpu.SemaphoreType.DMA,
            pltpu.SemaphoreType.DMA,
        ],
    )
    def agg(hp_hbm, pe_hbm, ew_hbm, out_hbm,
            r0, r1, r2, p0, p1, p2, e0, e1, e2, s0, s1, s2, acc_sh,
            lA, lB, lC, gA, gB, gC, sA, sB, sC, zsem):
        rows = [r0, r1, r2]
        peb = [p0, p1, p2]
        ewb = [e0, e1, e2]
        sidx = [s0, s1, s2]
        lsem = [lA, lB, lC]
        gsem = [gA, gB, gC]
        ssem = [sA, sB, sC]
        cid = lax.axis_index("c")
        sid = lax.axis_index("s")
        mask = (1 << PACK_SHIFT) - 1

        def fire_loads(b, k):
            pltpu.async_copy(pe_hbm.at[sid, b], peb[k].at[0], lsem[k])
            pltpu.async_copy(ew_hbm.at[sid, b], ewb[k].at[0], lsem[k])

        def wait_loads(k):
            pltpu.make_async_copy(pe_hbm.at[sid, 0], peb[k].at[0],
                                  lsem[k]).wait()
            pltpu.make_async_copy(ew_hbm.at[sid, 0], ewb[k].at[0],
                                  lsem[k]).wait()

        def unpack_fire_gather(k, base):
            def u(g, _):
                v = peb[k][0, pl.ds(g * LANES, LANES)]
                sidx[k][0, pl.ds(g * LANES, LANES)] = lax.shift_right_logical(
                    v, PACK_SHIFT)
                peb[k][0, pl.ds(g * LANES, LANES)] = (v & mask) + base
                return 0
            lax.fori_loop(0, AB // LANES, u, 0)
            pltpu.async_copy(hp_hbm.at[peb[k].at[0]], rows[k], gsem[k])

        def wait_gather(k):
            pltpu.make_async_copy(hp_hbm.at[peb[k].at[0]], rows[k],
                                  gsem[k]).wait()

        def scale(k):
            def sc1(g, _):
                ew16 = ewb[k][0, pl.ds(g * LANES, LANES)]
                for t in range(LANES):
                    s = ew16[t]
                    e = g * LANES + t
                    for j in range(8):
                        rows[k][e, pl.ds(j * LANES, LANES)] = (
                            rows[k][e, pl.ds(j * LANES, LANES)] * s)
                return 0
            lax.fori_loop(0, AB // LANES, sc1, 0)

        def fire_scatter(k):
            pltpu.async_copy(rows[k], acc_sh.at[sidx[k].at[0]], ssem[k],
                             add=True)

        def wait_scatter(k):
            pltpu.make_async_copy(rows[k], acc_sh.at[sidx[k].at[0]],
                                  ssem[k]).wait()

        for cl in range(cpc):
            chunk = cid * cpc + cl
            base = chunk * N

            def zf(i, _):
                for j in range(8):
                    r0[i, pl.ds(j * LANES, LANES)] = jnp.zeros(
                        (LANES,), jnp.float32)
                return 0
            lax.fori_loop(0, AB, zf, 0)

            def zc(q, _):
                pltpu.async_copy(r0.at[pl.ds(0, 64)],
                                 acc_sh.at[pl.ds(sid * NROW + q * 64, 64)],
                                 zsem)
                return 0
            lax.fori_loop(0, NROW // 64, zc, 0)

            def zw(q, _):
                pltpu.make_async_copy(r0.at[pl.ds(0, 64)],
                                      acc_sh.at[pl.ds(sid * NROW, 64)],
                                      zsem).wait()
                return 0
            lax.fori_loop(0, NROW // 64, zw, 0)
            plsc.subcore_barrier()

            fire_loads(0, 0)
            fire_loads(1, 1)
            wait_loads(0)
            unpack_fire_gather(0, base)
            fire_loads(2, 2)
            wait_loads(1)
            unpack_fire_gather(1, base)
            wait_gather(0)
            scale(0)
            fire_scatter(0)
            fire_loads(3, 0)
            wait_loads(2)
            unpack_fire_gather(2, base)
            wait_gather(1)
            scale(1)
            fire_scatter(1)

            def main(i, _):
                for j in range(3):
                    b = 2 + i * 3 + j
                    k = (2 + j) % 3
                    kp = j
                    kn = (1 + j) % 3
                    wait_scatter(kp)
                    fire_loads(b + 2, kn)
                    wait_loads(kp)
                    unpack_fire_gather(kp, base)
                    wait_gather(k)
                    scale(k)
                    fire_scatter(k)
                return 0
            lax.fori_loop(0, (NB_AGG - 4) // 3, main, 0)

            wait_scatter(0)
            wait_loads(0)
            unpack_fire_gather(0, base)
            wait_gather(2)
            scale(2)
            fire_scatter(2)
            wait_scatter(1)
            wait_gather(0)
            scale(0)
            fire_scatter(0)
            wait_scatter(2)
            wait_scatter(0)
            plsc.subcore_barrier()
            pltpu.sync_copy(acc_sh.at[pl.ds(sid * NROW, NROW)],
                            out_hbm.at[chunk, pl.ds(sid * NROW, NROW)])
    return agg


_agg4 = _make_agg(4)
_agg2 = _make_agg(2)


_RB = 1000
_GRID = N // _RB


def _tc1_body(x_ref, deg_ref, w1_ref, hp_ref, dinv_ref):
    deg = deg_ref[...]
    dinv = jnp.where(deg > 0, lax.rsqrt(deg), 0.0)
    p = jnp.dot(x_ref[...], w1_ref[...], preferred_element_type=jnp.float32)
    hp_ref[...] = p * dinv
    dinv_ref[...] = dinv


def _tc2_body(agg_ref, hp_ref, dinv_ref, b1_ref, w2_ref, out_ref):
    dinv = dinv_ref[...]
    h = jnp.maximum(dinv * (agg_ref[...] + hp_ref[...]) + b1_ref[...], 0.0)
    out_ref[...] = dinv * jnp.dot(h, w2_ref[...],
                                  preferred_element_type=jnp.float32)


def _tc3_body(agg_ref, hp_ref, dinv_ref, b2_ref, wd1_ref, bd1_ref, wd2_ref,
              bd2_ref, out_ref):
    dinv = dinv_ref[...]
    z = jnp.maximum(dinv * (agg_ref[...] + hp_ref[...]) + b2_ref[...], 0.0)
    d = jnp.maximum(jnp.dot(z, wd1_ref[...],
                            preferred_element_type=jnp.float32) + bd1_ref[...],
                    0.0)
    out_ref[...] = jax.nn.sigmoid(
        jnp.dot(d, wd2_ref[...], preferred_element_type=jnp.float32)
        + bd2_ref[...])


def _row_spec(ch):
    return pl.BlockSpec((_RB, ch), lambda i: (i, 0))


def _full_spec(r, c):
    return pl.BlockSpec((r, c), lambda i: (0, 0))


_tc1 = pl.pallas_call(
    _tc1_body,
    grid=(_GRID,),
    in_specs=[_row_spec(IN_CH), _row_spec(1), _full_spec(IN_CH, HID_CH)],
    out_specs=[_row_spec(HID_CH), _row_spec(1)],
    out_shape=[jax.ShapeDtypeStruct((N, HID_CH), jnp.float32),
               jax.ShapeDtypeStruct((N, 1), jnp.float32)],
)

_tc2 = pl.pallas_call(
    _tc2_body,
    grid=(_GRID,),
    in_specs=[_row_spec(HID_CH), _row_spec(HID_CH), _row_spec(1),
              _full_spec(1, HID_CH), _full_spec(HID_CH, LAT_CH)],
    out_specs=_row_spec(LAT_CH),
    out_shape=jax.ShapeDtypeStruct((N, LAT_CH), jnp.float32),
)

_tc3 = pl.pallas_call(
    _tc3_body,
    grid=(_GRID,),
    in_specs=[_row_spec(LAT_CH), _row_spec(LAT_CH), _row_spec(1),
              _full_spec(1, LAT_CH), _full_spec(LAT_CH, HID_CH),
              _full_spec(1, HID_CH), _full_spec(HID_CH, IN_CH),
              _full_spec(1, IN_CH)],
    out_specs=_row_spec(IN_CH),
    out_shape=jax.ShapeDtypeStruct((N, IN_CH), jnp.float32),
)


def _to_chunks(a, n_chunks):
    return a.reshape(N, n_chunks, 128).transpose(1, 0, 2).reshape(
        n_chunks * N, 128)


def _from_chunks(a, n_chunks):
    return a[:, :N, :].transpose(1, 0, 2).reshape(N, n_chunks * 128)


def kernel(x, edge_index, edge_weight, W1, b1, W2, b2, Wd1, bd1, Wd2, bd2):
    src = edge_index[0].astype(jnp.int32)
    dst = edge_index[1].astype(jnp.int32)
    ew = edge_weight.astype(jnp.float32)
    zpad_i = jnp.zeros((E_PAD_DEG - E,), jnp.int32)
    zpad_f = jnp.zeros((E_PAD_DEG - E,), jnp.float32)
    dst_d = jnp.concatenate([dst, zpad_i]).reshape(NCORE * NSUB, NB_DEG, EB)
    ew_d = jnp.concatenate([ew, zpad_f]).reshape(NCORE * NSUB, NB_DEG, EB)

    deg_part = _deg_kernel(dst_d, ew_d)
    deg = (deg_part[0, :N] + deg_part[1, :N] + 1.0).reshape(N, 1)

    pe = src + dst * (1 << PACK_SHIFT)
    pe_a = jnp.concatenate([pe, zpad_i[:E_PAD_AGG - E]]).reshape(
        NSUB, NB_AGG, AB)
    ew_a = jnp.concatenate([ew, zpad_f[:E_PAD_AGG - E]]).reshape(
        NSUB, NB_AGG, AB)

    h1p, dinv = _tc1(x, deg, W1)
    agg1 = _from_chunks(_agg4(_to_chunks(h1p, 4), pe_a, ew_a), 4)
    h2p = _tc2(agg1, h1p, dinv, b1.reshape(1, HID_CH), W2)
    agg2 = _from_chunks(_agg2(_to_chunks(h2p, 2), pe_a, ew_a), 2)
    x_hat = _tc3(agg2, h2p, dinv, b2.reshape(1, LAT_CH), Wd1,
                 bd1.reshape(1, HID_CH), Wd2, bd2.reshape(1, IN_CH))
    return x_hat
